# baseline (speedup 1.0000x reference)
import functools

import jax
import jax.numpy as jnp
from jax import lax
from jax.experimental import pallas as pl
from jax.experimental.pallas import tpu as pltpu
from jax.experimental.pallas import tpu_sc as plsc

N = 10000
C = 128
K = 32
OUT_C = 64

NPAD = 10240
NW = 32
QPW = NPAD // NW
QPC = 4
NCHUNK = QPW // QPC
IPC = QPC * K
TBL_W = 64
NBUF = 8
CPB = 8
BQ = CPB * QPC
Q_FAST = 448
Q_SLOW = 192
NB_FAST = Q_FAST // BQ
NB_SLOW = Q_SLOW // BQ
NCH_FAST = Q_FAST // QPC
NCH_SLOW = Q_SLOW // QPC
NKF = float(N * K)


def _prep_body(kft_ref, qft_ref, wt_ref, w1tp_ref, bias_ref, at_ref, bqt_ref):
    w1t = wt_ref[0:C, :]
    dwt = wt_ref[C : 2 * C, :] - w1t
    dn = (((0,), (0,)), ((), ()))
    at_ref[0:N, :] = lax.dot_general(
        kft_ref[...], w1tp_ref[...], dn, preferred_element_type=jnp.float32
    ).astype(jnp.bfloat16)
    at_ref[N:NPAD, :] = jnp.zeros((NPAD - N, OUT_C), jnp.bfloat16)
    bqt_ref[0:N, :] = (
        lax.dot_general(
            qft_ref[...], dwt, dn, preferred_element_type=jnp.float32
        )
        + bias_ref[...]
    )
    bqt_ref[N:NPAD, :] = jnp.zeros((NPAD - N, OUT_C), jnp.float32)


def _prep(kft, qft, wt, w1tp, bias2):
    return pl.pallas_call(
        _prep_body,
        out_shape=[
            jax.ShapeDtypeStruct((NPAD, TBL_W), jnp.bfloat16),
            jax.ShapeDtypeStruct((NPAD, OUT_C), jnp.float32),
        ],
    )(kft, qft, wt, w1tp, bias2)


def _sc_body(
    at_hbm, idx_hbm, bqt_hbm, out_hbm, part_hbm,
    idx_v, bq_v, part_v, rows, res, sems, osems,
):
    sid = lax.axis_index("s")
    cid = lax.axis_index("c")
    wid = sid * 2 + cid
    is_fast = cid == 0
    qbase = jnp.where(is_fast, sid * Q_FAST, 16 * Q_FAST + sid * Q_SLOW)
    nbp = jnp.where(is_fast, NB_FAST // 2, NB_SLOW // 2)
    nch = jnp.where(is_fast, NCH_FAST, NCH_SLOW)

    @pl.when(is_fast)
    def _():
        pltpu.sync_copy(
            idx_hbm.at[pl.ds(sid * NCH_FAST, NCH_FAST)],
            idx_v.at[pl.ds(0, NCH_FAST)],
        )

    if Q_SLOW > 0:

        @pl.when(jnp.logical_not(is_fast))
        def _():
            pltpu.sync_copy(
                idx_hbm.at[pl.ds(16 * NCH_FAST + sid * NCH_SLOW, NCH_SLOW)],
                idx_v.at[pl.ds(0, NCH_SLOW)],
            )

    @pl.when(is_fast)
    def _():
        pltpu.sync_copy(
            bqt_hbm.at[pl.ds(sid * Q_FAST, Q_FAST)], bq_v.at[pl.ds(0, Q_FAST)]
        )

    if Q_SLOW > 0:

        @pl.when(jnp.logical_not(is_fast))
        def _():
            pltpu.sync_copy(
                bqt_hbm.at[pl.ds(16 * Q_FAST + sid * Q_SLOW, Q_SLOW)],
                bq_v.at[pl.ds(0, Q_SLOW)],
            )

    for g0 in range(NBUF):

        @pl.when(g0 < nch)
        def _():
            pltpu.async_copy(at_hbm.at[idx_v.at[g0]], rows[g0], sems[g0])

    zeros = jnp.zeros((16,), jnp.float32)
    neg_inf = jnp.full((16,), -jnp.inf, jnp.float32)
    pos_inf = jnp.full((16,), jnp.inf, jnp.float32)

    def compute_chunk(blk, j, rbuf, rb, stats):
        for q in range(QPC):
            base = q * K

            def kbody(k, carry):
                lo = plsc.unpack(
                    rbuf[base + k, pl.ds(0, 32)], format=plsc.PackFormat.INTERLEAVED
                )
                hi = plsc.unpack(
                    rbuf[base + k, pl.ds(32, 32)], format=plsc.PackFormat.INTERLEAVED
                )
                vs = (lo[0], lo[1], hi[0], hi[1])
                out = []
                for cg in range(4):
                    s_a, q_a, mx_a, mn_a = carry[cg]
                    v = vs[cg]
                    out.append(
                        (
                            s_a + v,
                            q_a + v * v,
                            jnp.maximum(mx_a, v),
                            jnp.minimum(mn_a, v),
                        )
                    )
                return tuple(out)

            init = tuple((zeros, zeros, neg_inf, pos_inf) for _ in range(4))
            acc = lax.fori_loop(0, K, kbody, init, unroll=4)
            qrow = j * QPC + q
            wq = blk * BQ + qrow
            nstats = []
            for cg in range(4):
                s_a, q_a, mx_a, mn_a = acc[cg]
                res[rb][qrow, pl.ds(cg * 16, 16)] = mx_a
                res[rb][qrow, pl.ds(64 + cg * 16, 16)] = mn_a
                ss, sq, sx = stats[cg]
                bqv = bq_v[wq, pl.ds(cg * 16, 16)]
                nstats.append((ss + s_a, sq + q_a, sx + s_a * bqv))
            stats = tuple(nstats)
        return stats

    def blk_pair(bp, stats):
        for rb in range(2):
            blk = 2 * bp + rb

            @pl.when(blk >= 2)
            def _():
                pltpu.make_async_copy(
                    out_hbm.at[pl.ds(0, BQ)], res[rb], osems[rb]
                ).wait()

            for jj in range(CPB):
                g = blk * CPB + jj
                pltpu.make_async_copy(
                    at_hbm.at[idx_v.at[g]], rows[jj], sems[jj]
                ).wait()
                stats = compute_chunk(blk, jj, rows[jj], rb, stats)

                @pl.when(g + NBUF < nch)
                def _():
                    pltpu.async_copy(
                        at_hbm.at[idx_v.at[g + NBUF]], rows[jj], sems[jj]
                    )

            pltpu.async_copy(
                res[rb], out_hbm.at[pl.ds(qbase + blk * BQ, BQ)], osems[rb]
            )
        return stats

    stats0 = tuple((zeros, zeros, zeros) for _ in range(4))
    stats = lax.fori_loop(0, nbp, blk_pair, stats0)
    for cg in range(4):
        ss, sq, sx = stats[cg]
        part_v[0, pl.ds(cg * 16, 16)] = ss
        part_v[0, pl.ds(64 + cg * 16, 16)] = sq
        part_v[0, pl.ds(128 + cg * 16, 16)] = sx
        part_v[0, pl.ds(192 + cg * 16, 16)] = zeros
    pltpu.sync_copy(part_v, part_hbm.at[pl.ds(wid, 1)])
    for rb in range(2):

        @pl.when(nbp > 0)
        def _():
            pltpu.make_async_copy(
                out_hbm.at[pl.ds(0, BQ)], res[rb], osems[rb]
            ).wait()


def _sc_gather(at, idx3, bqt):
    mesh = plsc.VectorSubcoreMesh(
        core_axis_name="c", subcore_axis_name="s", num_cores=2, num_subcores=16
    )

    def body(
        at_hbm, idx_hbm, bqt_hbm, out_hbm, part_hbm,
        idx_v, bq_v, part_v, r0, r1, r2, r3, r4, r5, r6, r7, e0, e1,
        s0, s1, s2, s3, s4, s5, s6, s7, o0, o1,
    ):
        _sc_body(
            at_hbm, idx_hbm, bqt_hbm, out_hbm, part_hbm,
            idx_v, bq_v, part_v,
            (r0, r1, r2, r3, r4, r5, r6, r7), (e0, e1),
            (s0, s1, s2, s3, s4, s5, s6, s7), (o0, o1),
        )

    fn = pl.kernel(
        body,
        out_type=[
            jax.ShapeDtypeStruct((NPAD, 2 * OUT_C), jnp.float32),
            jax.ShapeDtypeStruct((NW, 4 * OUT_C), jnp.float32),
        ],
        mesh=mesh,
        compiler_params=pltpu.CompilerParams(
            use_tc_tiling_on_sc=False, needs_layout_passes=False
        ),
        scratch_types=[
            pltpu.VMEM((NCH_FAST, IPC), jnp.int32),
            pltpu.VMEM((Q_FAST, OUT_C), jnp.float32),
            pltpu.VMEM((1, 4 * OUT_C), jnp.float32),
            pltpu.VMEM((IPC, TBL_W), jnp.bfloat16),
            pltpu.VMEM((IPC, TBL_W), jnp.bfloat16),
            pltpu.VMEM((IPC, TBL_W), jnp.bfloat16),
            pltpu.VMEM((IPC, TBL_W), jnp.bfloat16),
            pltpu.VMEM((IPC, TBL_W), jnp.bfloat16),
            pltpu.VMEM((IPC, TBL_W), jnp.bfloat16),
            pltpu.VMEM((IPC, TBL_W), jnp.bfloat16),
            pltpu.VMEM((IPC, TBL_W), jnp.bfloat16),
            pltpu.VMEM((BQ, 2 * OUT_C), jnp.float32),
            pltpu.VMEM((BQ, 2 * OUT_C), jnp.float32),
            pltpu.SemaphoreType.DMA,
            pltpu.SemaphoreType.DMA,
            pltpu.SemaphoreType.DMA,
            pltpu.SemaphoreType.DMA,
            pltpu.SemaphoreType.DMA,
            pltpu.SemaphoreType.DMA,
            pltpu.SemaphoreType.DMA,
            pltpu.SemaphoreType.DMA,
            pltpu.SemaphoreType.DMA,
            pltpu.SemaphoreType.DMA,
        ],
    )
    return fn(at, idx3, bqt)


def _finish_body(r_ref, part_ref, bqt_ref, gamma_ref, beta_ref, out_ref):
    valid = (
        lax.broadcasted_iota(jnp.int32, (NPAD, 1), 0) < N
    ).astype(jnp.float32)
    b_g = bqt_ref[...] * valid

    sum_s = jnp.sum(part_ref[:, 0:OUT_C], axis=0, keepdims=True)
    sum_q = jnp.sum(part_ref[:, OUT_C : 2 * OUT_C], axis=0, keepdims=True)
    cross = jnp.sum(part_ref[:, 2 * OUT_C : 3 * OUT_C], axis=0, keepdims=True)
    sum_b = jnp.sum(b_g, axis=0, keepdims=True)
    sum_b2 = jnp.sum(b_g * b_g, axis=0, keepdims=True)

    mean = (sum_s + K * sum_b) * (1.0 / NKF)
    ey2 = (sum_q + 2.0 * cross + K * sum_b2) * (1.0 / NKF)
    var = ey2 - mean * mean
    scale = gamma_ref[...] * lax.rsqrt(var + 1e-5)
    shift = beta_ref[...] - scale * mean

    bq = bqt_ref[...]
    mx = r_ref[:, 0:OUT_C] + bq
    mn = r_ref[:, OUT_C : 2 * OUT_C] + bq
    m = jnp.where(scale >= 0.0, mx, mn)
    out_ref[...] = jnp.maximum(m * scale + shift, 0.0)


def _finish(r, part, bqt, gamma2, beta2):
    return pl.pallas_call(
        _finish_body,
        out_shape=jax.ShapeDtypeStruct((NPAD, OUT_C), jnp.float32),
    )(r, part, bqt, gamma2, beta2)


def kernel(query_feature, key_feature, key_ind, W, bias, gamma, beta):
    kft = key_feature[0]
    qft = query_feature[0]
    wt = W.T
    perm = []
    for g in range(2):
        for j in range(16):
            perm.extend([32 * g + j, 32 * g + 16 + j])
    w1tp = wt[:C][:, jnp.array(perm, dtype=jnp.int32)]
    bias2 = bias.reshape(1, OUT_C)
    gamma2 = gamma.reshape(1, OUT_C)
    beta2 = beta.reshape(1, OUT_C)

    at, bqt = _prep(kft, qft, wt, w1tp, bias2)

    idx = jnp.pad(
        key_ind[0].astype(jnp.int32), ((0, NPAD - N), (0, 0)), constant_values=N
    )
    idx2 = idx.reshape(NPAD * K // IPC, IPC)
    r, part = _sc_gather(at, idx2, bqt)

    out_t = _finish(r, part, bqt, gamma2, beta2)
    return out_t[:N].T[None]


# speedup vs baseline: 1.0282x; 1.0282x over previous
import functools

import jax
import jax.numpy as jnp
from jax import lax
from jax.experimental import pallas as pl
from jax.experimental.pallas import tpu as pltpu
from jax.experimental.pallas import tpu_sc as plsc

N = 10000
C = 128
K = 32
OUT_C = 64

NPAD = 10240
NW = 32
QPW = NPAD // NW
QPC = 4
NCHUNK = QPW // QPC
IPC = QPC * K
TBL_W = 64
NBUF = 4
CPB = 4
BQ = CPB * QPC
Q_FAST = 448
Q_SLOW = 192
NB_FAST = Q_FAST // BQ
NB_SLOW = Q_SLOW // BQ
NCH_FAST = Q_FAST // QPC
NCH_SLOW = Q_SLOW // QPC
NKF = float(N * K)


def _prep_body(kft_ref, qft_ref, wt_ref, w1tp_ref, bias_ref, at_ref, bqt_ref):
    w1t = wt_ref[0:C, :]
    dwt = wt_ref[C : 2 * C, :] - w1t
    dn = (((0,), (0,)), ((), ()))
    at_ref[0:N, :] = lax.dot_general(
        kft_ref[...], w1tp_ref[...], dn, preferred_element_type=jnp.float32
    ).astype(jnp.bfloat16)
    at_ref[N:NPAD, :] = jnp.zeros((NPAD - N, OUT_C), jnp.bfloat16)
    bqt_ref[0:N, :] = (
        lax.dot_general(
            qft_ref[...], dwt, dn, preferred_element_type=jnp.float32
        )
        + bias_ref[...]
    )
    bqt_ref[N:NPAD, :] = jnp.zeros((NPAD - N, OUT_C), jnp.float32)


def _prep(kft, qft, wt, w1tp, bias2):
    return pl.pallas_call(
        _prep_body,
        out_shape=[
            jax.ShapeDtypeStruct((NPAD, TBL_W), jnp.bfloat16),
            jax.ShapeDtypeStruct((NPAD, OUT_C), jnp.float32),
        ],
    )(kft, qft, wt, w1tp, bias2)


def _sc_body(
    at_hbm, idx_hbm, bqt_hbm, out_hbm, part_hbm,
    idx_v, bq_v, part_v, rows, res, sems, osems,
):
    sid = lax.axis_index("s")
    cid = lax.axis_index("c")
    wid = sid * 2 + cid
    is_fast = cid == 0
    qbase = jnp.where(is_fast, sid * Q_FAST, 16 * Q_FAST + sid * Q_SLOW)
    nbp = jnp.where(is_fast, NB_FAST // 2, NB_SLOW // 2)
    nch = jnp.where(is_fast, NCH_FAST, NCH_SLOW)

    @pl.when(is_fast)
    def _():
        pltpu.sync_copy(
            idx_hbm.at[pl.ds(sid * NCH_FAST, NCH_FAST)],
            idx_v.at[pl.ds(0, NCH_FAST)],
        )

    if Q_SLOW > 0:

        @pl.when(jnp.logical_not(is_fast))
        def _():
            pltpu.sync_copy(
                idx_hbm.at[pl.ds(16 * NCH_FAST + sid * NCH_SLOW, NCH_SLOW)],
                idx_v.at[pl.ds(0, NCH_SLOW)],
            )

    @pl.when(is_fast)
    def _():
        pltpu.sync_copy(
            bqt_hbm.at[pl.ds(sid * Q_FAST, Q_FAST)], bq_v.at[pl.ds(0, Q_FAST)]
        )

    if Q_SLOW > 0:

        @pl.when(jnp.logical_not(is_fast))
        def _():
            pltpu.sync_copy(
                bqt_hbm.at[pl.ds(16 * Q_FAST + sid * Q_SLOW, Q_SLOW)],
                bq_v.at[pl.ds(0, Q_SLOW)],
            )

    for g0 in range(NBUF):

        @pl.when(g0 < nch)
        def _():
            pltpu.async_copy(at_hbm.at[idx_v.at[g0]], rows[g0], sems[g0])

    zeros = jnp.zeros((16,), jnp.float32)
    neg_inf = jnp.full((16,), -jnp.inf, jnp.float32)
    pos_inf = jnp.full((16,), jnp.inf, jnp.float32)

    def compute_chunk(blk, j, rbuf, rb, stats):
        for q in range(QPC):
            base = q * K

            def kbody(k, carry):
                lo = plsc.unpack(
                    rbuf[base + k, pl.ds(0, 32)], format=plsc.PackFormat.INTERLEAVED
                )
                hi = plsc.unpack(
                    rbuf[base + k, pl.ds(32, 32)], format=plsc.PackFormat.INTERLEAVED
                )
                vs = (lo[0], lo[1], hi[0], hi[1])
                out = []
                for cg in range(4):
                    s_a, q_a, mx_a, mn_a = carry[cg]
                    v = vs[cg]
                    out.append(
                        (
                            s_a + v,
                            q_a + v * v,
                            jnp.maximum(mx_a, v),
                            jnp.minimum(mn_a, v),
                        )
                    )
                return tuple(out)

            init = tuple((zeros, zeros, neg_inf, pos_inf) for _ in range(4))
            acc = lax.fori_loop(0, K, kbody, init, unroll=4)
            qrow = j * QPC + q
            wq = blk * BQ + qrow
            nstats = []
            for cg in range(4):
                s_a, q_a, mx_a, mn_a = acc[cg]
                res[rb][qrow, pl.ds(cg * 16, 16)] = mx_a
                res[rb][qrow, pl.ds(64 + cg * 16, 16)] = mn_a
                ss, sq, sx = stats[cg]
                bqv = bq_v[wq, pl.ds(cg * 16, 16)]
                nstats.append((ss + s_a, sq + q_a, sx + s_a * bqv))
            stats = tuple(nstats)
        return stats

    def blk_pair(bp, stats):
        for rb in range(2):
            blk = 2 * bp + rb

            @pl.when(blk >= 2)
            def _():
                pltpu.make_async_copy(
                    out_hbm.at[pl.ds(0, BQ)], res[rb], osems[rb]
                ).wait()

            for jj in range(CPB):
                g = blk * CPB + jj
                pltpu.make_async_copy(
                    at_hbm.at[idx_v.at[g]], rows[jj], sems[jj]
                ).wait()
                stats = compute_chunk(blk, jj, rows[jj], rb, stats)

                @pl.when(g + NBUF < nch)
                def _():
                    pltpu.async_copy(
                        at_hbm.at[idx_v.at[g + NBUF]], rows[jj], sems[jj]
                    )

            pltpu.async_copy(
                res[rb], out_hbm.at[pl.ds(qbase + blk * BQ, BQ)], osems[rb]
            )
        return stats

    stats0 = tuple((zeros, zeros, zeros) for _ in range(4))
    stats = lax.fori_loop(0, nbp, blk_pair, stats0)
    for cg in range(4):
        ss, sq, sx = stats[cg]
        part_v[0, pl.ds(cg * 16, 16)] = ss
        part_v[0, pl.ds(64 + cg * 16, 16)] = sq
        part_v[0, pl.ds(128 + cg * 16, 16)] = sx
        part_v[0, pl.ds(192 + cg * 16, 16)] = zeros
    pltpu.sync_copy(part_v, part_hbm.at[pl.ds(wid, 1)])
    for rb in range(2):

        @pl.when(nbp > 0)
        def _():
            pltpu.make_async_copy(
                out_hbm.at[pl.ds(0, BQ)], res[rb], osems[rb]
            ).wait()


def _sc_gather(at, idx3, bqt):
    mesh = plsc.VectorSubcoreMesh(
        core_axis_name="c", subcore_axis_name="s", num_cores=2, num_subcores=16
    )

    def body(
        at_hbm, idx_hbm, bqt_hbm, out_hbm, part_hbm,
        idx_v, bq_v, part_v, r0, r1, r2, r3, e0, e1,
        s0, s1, s2, s3, o0, o1,
    ):
        _sc_body(
            at_hbm, idx_hbm, bqt_hbm, out_hbm, part_hbm,
            idx_v, bq_v, part_v,
            (r0, r1, r2, r3), (e0, e1), (s0, s1, s2, s3), (o0, o1),
        )

    fn = pl.kernel(
        body,
        out_type=[
            jax.ShapeDtypeStruct((NPAD, 2 * OUT_C), jnp.float32),
            jax.ShapeDtypeStruct((NW, 4 * OUT_C), jnp.float32),
        ],
        mesh=mesh,
        compiler_params=pltpu.CompilerParams(
            use_tc_tiling_on_sc=False, needs_layout_passes=False
        ),
        scratch_types=[
            pltpu.VMEM((NCH_FAST, IPC), jnp.int32),
            pltpu.VMEM((Q_FAST, OUT_C), jnp.float32),
            pltpu.VMEM((1, 4 * OUT_C), jnp.float32),
            pltpu.VMEM((IPC, TBL_W), jnp.bfloat16),
            pltpu.VMEM((IPC, TBL_W), jnp.bfloat16),
            pltpu.VMEM((IPC, TBL_W), jnp.bfloat16),
            pltpu.VMEM((IPC, TBL_W), jnp.bfloat16),
            pltpu.VMEM((BQ, 2 * OUT_C), jnp.float32),
            pltpu.VMEM((BQ, 2 * OUT_C), jnp.float32),
            pltpu.SemaphoreType.DMA,
            pltpu.SemaphoreType.DMA,
            pltpu.SemaphoreType.DMA,
            pltpu.SemaphoreType.DMA,
            pltpu.SemaphoreType.DMA,
            pltpu.SemaphoreType.DMA,
        ],
    )
    return fn(at, idx3, bqt)


def _finish_body(r_ref, part_ref, bqt_ref, gamma_ref, beta_ref, out_ref):
    valid = (
        lax.broadcasted_iota(jnp.int32, (NPAD, 1), 0) < N
    ).astype(jnp.float32)
    b_g = bqt_ref[...] * valid

    sum_s = jnp.sum(part_ref[:, 0:OUT_C], axis=0, keepdims=True)
    sum_q = jnp.sum(part_ref[:, OUT_C : 2 * OUT_C], axis=0, keepdims=True)
    cross = jnp.sum(part_ref[:, 2 * OUT_C : 3 * OUT_C], axis=0, keepdims=True)
    sum_b = jnp.sum(b_g, axis=0, keepdims=True)
    sum_b2 = jnp.sum(b_g * b_g, axis=0, keepdims=True)

    mean = (sum_s + K * sum_b) * (1.0 / NKF)
    ey2 = (sum_q + 2.0 * cross + K * sum_b2) * (1.0 / NKF)
    var = ey2 - mean * mean
    scale = gamma_ref[...] * lax.rsqrt(var + 1e-5)
    shift = beta_ref[...] - scale * mean

    bq = bqt_ref[...]
    mx = r_ref[:, 0:OUT_C] + bq
    mn = r_ref[:, OUT_C : 2 * OUT_C] + bq
    m = jnp.where(scale >= 0.0, mx, mn)
    out_ref[...] = jnp.maximum(m * scale + shift, 0.0)


def _finish(r, part, bqt, gamma2, beta2):
    return pl.pallas_call(
        _finish_body,
        out_shape=jax.ShapeDtypeStruct((NPAD, OUT_C), jnp.float32),
    )(r, part, bqt, gamma2, beta2)


def kernel(query_feature, key_feature, key_ind, W, bias, gamma, beta):
    kft = key_feature[0]
    qft = query_feature[0]
    wt = W.T
    perm = []
    for g in range(2):
        for j in range(16):
            perm.extend([32 * g + j, 32 * g + 16 + j])
    w1tp = wt[:C][:, jnp.array(perm, dtype=jnp.int32)]
    bias2 = bias.reshape(1, OUT_C)
    gamma2 = gamma.reshape(1, OUT_C)
    beta2 = beta.reshape(1, OUT_C)

    at, bqt = _prep(kft, qft, wt, w1tp, bias2)

    idx = jnp.pad(
        key_ind[0].astype(jnp.int32), ((0, NPAD - N), (0, 0)), constant_values=N
    )
    idx2 = idx.reshape(NPAD * K // IPC, IPC)
    r, part = _sc_gather(at, idx2, bqt)

    out_t = _finish(r, part, bqt, gamma2, beta2)
    return out_t[:N].T[None]
